# baseline (speedup 1.0000x reference)
import functools
import jax
import jax.numpy as jnp
from jax import lax
from jax.experimental import pallas as pl
from jax.experimental.pallas import tpu as pltpu
from jax.experimental.pallas import tpu_sc as plsc

N_NODES = 10000
N_PAD = 10240
N_EDGES = 320000
D = 128

NC = 2
NS = 16
NW = NC * NS
EDGES_PER_W = N_EDGES // NW
EDGE_BLK = 80
N_CHUNKS = EDGES_PER_W // EDGE_BLK
NBUF = 3
N_RING = N_CHUNKS // NBUF
N_TAIL = N_CHUNKS - NBUF * N_RING
ROWS_PER_S = N_PAD // NS


@functools.lru_cache(maxsize=None)
def _make_sc_scatter(width):
    mesh = plsc.VectorSubcoreMesh(core_axis_name="c", subcore_axis_name="s")

    def body(y_hbm, ei_hbm, zeros_hbm, p_hbm, idx0, idx1, idx2,
             rows0, rows1, rows2, acc_s, g0, g1, g2, s0, s1, s2):
        c = lax.axis_index("c")
        s = lax.axis_index("s")
        wid = s * NC + c
        rbase = s * ROWS_PER_S
        pltpu.sync_copy(zeros_hbm.at[pl.ds(rbase, ROWS_PER_S)],
                        acc_s.at[pl.ds(rbase, ROWS_PER_S)])
        plsc.subcore_barrier()

        idx = (idx0, idx1, idx2)
        rows = (rows0, rows1, rows2)
        gsem = (g0, g1, g2)
        ssem = (s0, s1, s2)

        def ring(i, carry):
            for k in range(NBUF):
                @pl.when(i > 0)
                def _(k=k):
                    pltpu.make_async_copy(
                        rows[k], acc_s.at[idx[k].at[1]], ssem[k]).wait()
                pltpu.sync_copy(ei_hbm.at[wid, i * NBUF + k], idx[k])
                pltpu.async_copy(y_hbm.at[idx[k].at[0]], rows[k], gsem[k])
            for k in range(NBUF):
                pltpu.make_async_copy(
                    y_hbm.at[idx[k].at[0]], rows[k], gsem[k]).wait()
                pltpu.async_copy(
                    rows[k], acc_s.at[idx[k].at[1]], ssem[k], add=True)
            return carry

        lax.fori_loop(0, N_RING, ring, 0)
        for k in range(N_TAIL):
            pltpu.make_async_copy(
                rows[k], acc_s.at[idx[k].at[1]], ssem[k]).wait()
            pltpu.sync_copy(ei_hbm.at[wid, N_RING * NBUF + k], idx[k])
            pltpu.async_copy(y_hbm.at[idx[k].at[0]], rows[k], gsem[k])
        for k in range(N_TAIL):
            pltpu.make_async_copy(
                y_hbm.at[idx[k].at[0]], rows[k], gsem[k]).wait()
            pltpu.async_copy(
                rows[k], acc_s.at[idx[k].at[1]], ssem[k], add=True)
        for k in range(N_TAIL, NBUF):
            pltpu.make_async_copy(
                rows[k], acc_s.at[idx[k].at[1]], ssem[k]).wait()
        for k in range(N_TAIL):
            pltpu.make_async_copy(
                rows[k], acc_s.at[idx[k].at[1]], ssem[k]).wait()
        plsc.subcore_barrier()

        pltpu.sync_copy(acc_s.at[pl.ds(rbase, ROWS_PER_S)],
                        p_hbm.at[c, pl.ds(rbase, ROWS_PER_S)])

    return pl.kernel(
        body,
        out_type=jax.ShapeDtypeStruct((NC, N_PAD, width), jnp.float32),
        mesh=mesh,
        scratch_types=(
            [pltpu.VMEM((2, EDGE_BLK), jnp.int32) for _ in range(NBUF)]
            + [pltpu.VMEM((EDGE_BLK, width), jnp.float32)
               for _ in range(NBUF)]
            + [pltpu.VMEM_SHARED((N_PAD, width), jnp.float32)]
            + [pltpu.SemaphoreType.DMA] * (2 * NBUF)
        ),
    )


@functools.lru_cache(maxsize=None)
def _make_sc_deg():
    mesh = plsc.VectorSubcoreMesh(core_axis_name="c", subcore_axis_name="s")

    def body(dst_hbm, zeros_hbm, ones_hbm, degp_hbm, dst_i, ones_v, acc_s):
        c = lax.axis_index("c")
        s = lax.axis_index("s")
        wid = s * NC + c
        rbase = s * ROWS_PER_S
        pltpu.sync_copy(zeros_hbm.at[pl.ds(rbase, ROWS_PER_S)],
                        acc_s.at[pl.ds(rbase, ROWS_PER_S)])
        pltpu.sync_copy(ones_hbm, ones_v)
        pltpu.sync_copy(dst_hbm.at[wid], dst_i)
        plsc.subcore_barrier()

        def chunk(i, carry):
            pltpu.sync_copy(ones_v, acc_s.at[dst_i.at[i]], add=True)
            return carry

        lax.fori_loop(0, N_CHUNKS, chunk, 0)
        plsc.subcore_barrier()

        pltpu.sync_copy(acc_s.at[pl.ds(rbase, ROWS_PER_S)],
                        degp_hbm.at[c, pl.ds(rbase, ROWS_PER_S)])

    return pl.kernel(
        body,
        out_type=jax.ShapeDtypeStruct((NC, N_PAD, D), jnp.float32),
        mesh=mesh,
        scratch_types=[
            pltpu.VMEM((N_CHUNKS, EDGE_BLK), jnp.int32),
            pltpu.VMEM((EDGE_BLK, D), jnp.float32),
            pltpu.VMEM_SHARED((N_PAD, D), jnp.float32),
        ],
    )


ROW_BLK = 640
N_ROW_BLKS = N_PAD // ROW_BLK


def _tc_in_body(x_ref, wl_ref, wr_ref, b_ref, y_ref, z_ref):
    x = x_ref[...]
    y_ref[...] = jnp.dot(x, wl_ref[...], preferred_element_type=jnp.float32)
    z_ref[...] = (jnp.dot(x, wr_ref[...], preferred_element_type=jnp.float32)
                  + b_ref[...])


def _tc_in(x, W_l, W_r, b):
    return pl.pallas_call(
        _tc_in_body,
        grid=(N_ROW_BLKS,),
        in_specs=[
            pl.BlockSpec((ROW_BLK, D), lambda i: (i, 0)),
            pl.BlockSpec((D, D), lambda i: (0, 0)),
            pl.BlockSpec((D, D), lambda i: (0, 0)),
            pl.BlockSpec((1, D), lambda i: (0, 0)),
        ],
        out_specs=[
            pl.BlockSpec((ROW_BLK, D), lambda i: (i, 0)),
            pl.BlockSpec((ROW_BLK, D), lambda i: (i, 0)),
        ],
        out_shape=[
            jax.ShapeDtypeStruct((N_PAD, D), jnp.float32),
            jax.ShapeDtypeStruct((N_PAD, D), jnp.float32),
        ],
    )(x, W_l, W_r, b.reshape(1, D))


def _tc_mid_body(p0_ref, p1_ref, d0_ref, d1_ref, z_ref, wl_ref, wr_ref,
                 b_ref, y_ref, z2_ref, r_ref):
    deg = d0_ref[:, 0:1] + d1_ref[:, 0:1]
    r = 1.0 / jnp.maximum(deg, 1.0)
    agg = p0_ref[...] + p1_ref[...]
    h = jnp.maximum(agg * r + z_ref[...], 0.0)
    y_ref[...] = jnp.dot(h, wl_ref[...], preferred_element_type=jnp.float32)
    z2_ref[...] = (jnp.dot(h, wr_ref[...], preferred_element_type=jnp.float32)
                   + b_ref[...])
    r_ref[...] = jnp.broadcast_to(r, (ROW_BLK, 8))


def _tc_mid(p0, p1, d0, d1, z1, W_l, W_r, b):
    return pl.pallas_call(
        _tc_mid_body,
        grid=(N_ROW_BLKS,),
        in_specs=[
            pl.BlockSpec((ROW_BLK, D), lambda i: (i, 0)),
            pl.BlockSpec((ROW_BLK, D), lambda i: (i, 0)),
            pl.BlockSpec((ROW_BLK, D), lambda i: (i, 0)),
            pl.BlockSpec((ROW_BLK, D), lambda i: (i, 0)),
            pl.BlockSpec((ROW_BLK, D), lambda i: (i, 0)),
            pl.BlockSpec((D, D), lambda i: (0, 0)),
            pl.BlockSpec((D, D), lambda i: (0, 0)),
            pl.BlockSpec((1, D), lambda i: (0, 0)),
        ],
        out_specs=[
            pl.BlockSpec((ROW_BLK, D), lambda i: (i, 0)),
            pl.BlockSpec((ROW_BLK, D), lambda i: (i, 0)),
            pl.BlockSpec((ROW_BLK, 8), lambda i: (i, 0)),
        ],
        out_shape=[
            jax.ShapeDtypeStruct((N_PAD, D), jnp.float32),
            jax.ShapeDtypeStruct((N_PAD, D), jnp.float32),
            jax.ShapeDtypeStruct((N_PAD, 8), jnp.float32),
        ],
    )(p0, p1, d0, d1, z1, W_l, W_r, b.reshape(1, D))


def _tc_out_body(q0_ref, q1_ref, r_ref, z_ref, o_ref):
    r = r_ref[...][:, 0:1]
    o_ref[...] = (q0_ref[...] + q1_ref[...]) * r + z_ref[...]


def _tc_out(q0, q1, r, z2):
    return pl.pallas_call(
        _tc_out_body,
        grid=(N_ROW_BLKS,),
        in_specs=[
            pl.BlockSpec((ROW_BLK, D), lambda i: (i, 0)),
            pl.BlockSpec((ROW_BLK, D), lambda i: (i, 0)),
            pl.BlockSpec((ROW_BLK, 8), lambda i: (i, 0)),
            pl.BlockSpec((ROW_BLK, D), lambda i: (i, 0)),
        ],
        out_specs=pl.BlockSpec((ROW_BLK, D), lambda i: (i, 0)),
        out_shape=jax.ShapeDtypeStruct((N_PAD, D), jnp.float32),
    )(q0, q1, r, z2)


@jax.jit
def kernel(x, edge_index, W1_l, W1_r, b1, W2_l, W2_r, b2):
    src = edge_index[0].astype(jnp.int32).reshape(NW, N_CHUNKS, EDGE_BLK)
    dst = edge_index[1].astype(jnp.int32).reshape(NW, N_CHUNKS, EDGE_BLK)
    ei = jnp.stack([src, dst], axis=2)
    zeros = jnp.zeros((N_PAD, D), jnp.float32)
    ones = jnp.ones((EDGE_BLK, D), jnp.float32)
    xp = jnp.pad(x, ((0, N_PAD - N_NODES), (0, 0)))

    degp = _make_sc_deg()(dst, zeros, ones)
    y1, z1 = _tc_in(xp, W1_l, W1_r, b1)
    p = _make_sc_scatter(D)(y1, ei, zeros)
    y2, z2, r = _tc_mid(p[0], p[1], degp[0], degp[1], z1, W2_l, W2_r, b2)
    q = _make_sc_scatter(D)(y2, ei, zeros)
    return _tc_out(q[0], q[1], r, z2)[:N_NODES]


# speedup vs baseline: 1.0734x; 1.0734x over previous
import functools
import jax
import jax.numpy as jnp
from jax import lax
from jax.experimental import pallas as pl
from jax.experimental.pallas import tpu as pltpu
from jax.experimental.pallas import tpu_sc as plsc

N_NODES = 10000
N_PAD = 10240
N_EDGES = 320000
D = 128

NC = 2
NS = 16
NW = NC * NS
EDGES_PER_W = N_EDGES // NW
EDGE_BLK = 100
N_CHUNKS = EDGES_PER_W // EDGE_BLK
NBUF = 3
N_RING = N_CHUNKS // NBUF
N_TAIL = N_CHUNKS - NBUF * N_RING
ROWS_PER_S = N_PAD // NS


@functools.lru_cache(maxsize=None)
def _make_sc_scatter(width):
    mesh = plsc.VectorSubcoreMesh(core_axis_name="c", subcore_axis_name="s")

    def body(y_hbm, ei_hbm, zeros_hbm, p_hbm, idx0, idx1, idx2,
             rows0, rows1, rows2, acc_s, g0, g1, g2, s0, s1, s2):
        c = lax.axis_index("c")
        s = lax.axis_index("s")
        wid = s * NC + c
        rbase = s * ROWS_PER_S
        pltpu.sync_copy(zeros_hbm.at[pl.ds(rbase, ROWS_PER_S)],
                        acc_s.at[pl.ds(rbase, ROWS_PER_S)])
        plsc.subcore_barrier()

        idx = (idx0, idx1, idx2)
        rows = (rows0, rows1, rows2)
        gsem = (g0, g1, g2)
        ssem = (s0, s1, s2)

        def ring(i, carry):
            for k in range(NBUF):
                @pl.when(i > 0)
                def _(k=k):
                    pltpu.make_async_copy(
                        rows[k], acc_s.at[idx[k].at[1]], ssem[k]).wait()
                pltpu.sync_copy(ei_hbm.at[wid, i * NBUF + k], idx[k])
                pltpu.async_copy(y_hbm.at[idx[k].at[0]], rows[k], gsem[k])
            for k in range(NBUF):
                pltpu.make_async_copy(
                    y_hbm.at[idx[k].at[0]], rows[k], gsem[k]).wait()
                pltpu.async_copy(
                    rows[k], acc_s.at[idx[k].at[1]], ssem[k], add=True)
            return carry

        lax.fori_loop(0, N_RING, ring, 0)
        for k in range(N_TAIL):
            pltpu.make_async_copy(
                rows[k], acc_s.at[idx[k].at[1]], ssem[k]).wait()
            pltpu.sync_copy(ei_hbm.at[wid, N_RING * NBUF + k], idx[k])
            pltpu.async_copy(y_hbm.at[idx[k].at[0]], rows[k], gsem[k])
        for k in range(N_TAIL):
            pltpu.make_async_copy(
                y_hbm.at[idx[k].at[0]], rows[k], gsem[k]).wait()
            pltpu.async_copy(
                rows[k], acc_s.at[idx[k].at[1]], ssem[k], add=True)
        for k in range(N_TAIL, NBUF):
            pltpu.make_async_copy(
                rows[k], acc_s.at[idx[k].at[1]], ssem[k]).wait()
        for k in range(N_TAIL):
            pltpu.make_async_copy(
                rows[k], acc_s.at[idx[k].at[1]], ssem[k]).wait()
        plsc.subcore_barrier()

        pltpu.sync_copy(acc_s.at[pl.ds(rbase, ROWS_PER_S)],
                        p_hbm.at[c, pl.ds(rbase, ROWS_PER_S)])

    return pl.kernel(
        body,
        out_type=jax.ShapeDtypeStruct((NC, N_PAD, width), jnp.float32),
        mesh=mesh,
        scratch_types=(
            [pltpu.VMEM((2, EDGE_BLK), jnp.int32) for _ in range(NBUF)]
            + [pltpu.VMEM((EDGE_BLK, width), jnp.float32)
               for _ in range(NBUF)]
            + [pltpu.VMEM_SHARED((N_PAD, width), jnp.float32)]
            + [pltpu.SemaphoreType.DMA] * (2 * NBUF)
        ),
    )


@functools.lru_cache(maxsize=None)
def _make_sc_deg():
    mesh = plsc.VectorSubcoreMesh(core_axis_name="c", subcore_axis_name="s")

    def body(dst_hbm, zeros_hbm, ones_hbm, degp_hbm, dst_i, ones_v, acc_s):
        c = lax.axis_index("c")
        s = lax.axis_index("s")
        wid = s * NC + c
        rbase = s * ROWS_PER_S
        pltpu.sync_copy(zeros_hbm.at[pl.ds(rbase, ROWS_PER_S)],
                        acc_s.at[pl.ds(rbase, ROWS_PER_S)])
        pltpu.sync_copy(ones_hbm, ones_v)
        pltpu.sync_copy(dst_hbm.at[wid], dst_i)
        plsc.subcore_barrier()

        def chunk(i, carry):
            pltpu.sync_copy(ones_v, acc_s.at[dst_i.at[i]], add=True)
            return carry

        lax.fori_loop(0, N_CHUNKS, chunk, 0)
        plsc.subcore_barrier()

        pltpu.sync_copy(acc_s.at[pl.ds(rbase, ROWS_PER_S)],
                        degp_hbm.at[c, pl.ds(rbase, ROWS_PER_S)])

    return pl.kernel(
        body,
        out_type=jax.ShapeDtypeStruct((NC, N_PAD, D), jnp.float32),
        mesh=mesh,
        scratch_types=[
            pltpu.VMEM((N_CHUNKS, EDGE_BLK), jnp.int32),
            pltpu.VMEM((EDGE_BLK, D), jnp.float32),
            pltpu.VMEM_SHARED((N_PAD, D), jnp.float32),
        ],
    )


ROW_BLK = 640
N_ROW_BLKS = N_PAD // ROW_BLK


def _tc_in_body(x_ref, wl_ref, wr_ref, b_ref, y_ref, z_ref):
    x = x_ref[...]
    y_ref[...] = jnp.dot(x, wl_ref[...], preferred_element_type=jnp.float32)
    z_ref[...] = (jnp.dot(x, wr_ref[...], preferred_element_type=jnp.float32)
                  + b_ref[...])


def _tc_in(x, W_l, W_r, b):
    return pl.pallas_call(
        _tc_in_body,
        grid=(N_ROW_BLKS,),
        in_specs=[
            pl.BlockSpec((ROW_BLK, D), lambda i: (i, 0)),
            pl.BlockSpec((D, D), lambda i: (0, 0)),
            pl.BlockSpec((D, D), lambda i: (0, 0)),
            pl.BlockSpec((1, D), lambda i: (0, 0)),
        ],
        out_specs=[
            pl.BlockSpec((ROW_BLK, D), lambda i: (i, 0)),
            pl.BlockSpec((ROW_BLK, D), lambda i: (i, 0)),
        ],
        out_shape=[
            jax.ShapeDtypeStruct((N_PAD, D), jnp.float32),
            jax.ShapeDtypeStruct((N_PAD, D), jnp.float32),
        ],
    )(x, W_l, W_r, b.reshape(1, D))


def _tc_mid_body(p0_ref, p1_ref, d0_ref, d1_ref, z_ref, wl_ref, wr_ref,
                 b_ref, y_ref, z2_ref, r_ref):
    deg = d0_ref[:, 0:1] + d1_ref[:, 0:1]
    r = 1.0 / jnp.maximum(deg, 1.0)
    agg = p0_ref[...] + p1_ref[...]
    h = jnp.maximum(agg * r + z_ref[...], 0.0)
    y_ref[...] = jnp.dot(h, wl_ref[...], preferred_element_type=jnp.float32)
    z2_ref[...] = (jnp.dot(h, wr_ref[...], preferred_element_type=jnp.float32)
                   + b_ref[...])
    r_ref[...] = jnp.broadcast_to(r, (ROW_BLK, 8))


def _tc_mid(p0, p1, d0, d1, z1, W_l, W_r, b):
    return pl.pallas_call(
        _tc_mid_body,
        grid=(N_ROW_BLKS,),
        in_specs=[
            pl.BlockSpec((ROW_BLK, D), lambda i: (i, 0)),
            pl.BlockSpec((ROW_BLK, D), lambda i: (i, 0)),
            pl.BlockSpec((ROW_BLK, D), lambda i: (i, 0)),
            pl.BlockSpec((ROW_BLK, D), lambda i: (i, 0)),
            pl.BlockSpec((ROW_BLK, D), lambda i: (i, 0)),
            pl.BlockSpec((D, D), lambda i: (0, 0)),
            pl.BlockSpec((D, D), lambda i: (0, 0)),
            pl.BlockSpec((1, D), lambda i: (0, 0)),
        ],
        out_specs=[
            pl.BlockSpec((ROW_BLK, D), lambda i: (i, 0)),
            pl.BlockSpec((ROW_BLK, D), lambda i: (i, 0)),
            pl.BlockSpec((ROW_BLK, 8), lambda i: (i, 0)),
        ],
        out_shape=[
            jax.ShapeDtypeStruct((N_PAD, D), jnp.float32),
            jax.ShapeDtypeStruct((N_PAD, D), jnp.float32),
            jax.ShapeDtypeStruct((N_PAD, 8), jnp.float32),
        ],
    )(p0, p1, d0, d1, z1, W_l, W_r, b.reshape(1, D))


def _tc_out_body(q0_ref, q1_ref, r_ref, z_ref, o_ref):
    r = r_ref[...][:, 0:1]
    o_ref[...] = (q0_ref[...] + q1_ref[...]) * r + z_ref[...]


def _tc_out(q0, q1, r, z2):
    return pl.pallas_call(
        _tc_out_body,
        grid=(N_ROW_BLKS,),
        in_specs=[
            pl.BlockSpec((ROW_BLK, D), lambda i: (i, 0)),
            pl.BlockSpec((ROW_BLK, D), lambda i: (i, 0)),
            pl.BlockSpec((ROW_BLK, 8), lambda i: (i, 0)),
            pl.BlockSpec((ROW_BLK, D), lambda i: (i, 0)),
        ],
        out_specs=pl.BlockSpec((ROW_BLK, D), lambda i: (i, 0)),
        out_shape=jax.ShapeDtypeStruct((N_PAD, D), jnp.float32),
    )(q0, q1, r, z2)


@jax.jit
def kernel(x, edge_index, W1_l, W1_r, b1, W2_l, W2_r, b2):
    src = edge_index[0].astype(jnp.int32).reshape(NW, N_CHUNKS, EDGE_BLK)
    dst = edge_index[1].astype(jnp.int32).reshape(NW, N_CHUNKS, EDGE_BLK)
    ei = jnp.stack([src, dst], axis=2)
    zeros = jnp.zeros((N_PAD, D), jnp.float32)
    ones = jnp.ones((EDGE_BLK, D), jnp.float32)
    xp = jnp.pad(x, ((0, N_PAD - N_NODES), (0, 0)))

    degp = _make_sc_deg()(dst, zeros, ones)
    y1, z1 = _tc_in(xp, W1_l, W1_r, b1)
    p = _make_sc_scatter(D)(y1, ei, zeros)
    y2, z2, r = _tc_mid(p[0], p[1], degp[0], degp[1], z1, W2_l, W2_r, b2)
    q = _make_sc_scatter(D)(y2, ei, zeros)
    return _tc_out(q[0], q[1], r, z2)[:N_NODES]
